# Initial kernel scaffold; baseline (speedup 1.0000x reference)
import functools

import jax
import jax.numpy as jnp
from jax import lax
from jax.experimental import pallas as pl
from jax.experimental.pallas import tpu as pltpu
from jax.experimental.pallas import tpu_sc as plsc

NC = 2
NS = 16
LANES = 16
NW = NC * NS

CHUNK = 80


@functools.partial(jax.jit, static_argnames=("n_nodes",))
def _sc_agg(x, src2, dst2, *, n_nodes):
    n, d = x.shape
    n_chunks = src2.shape[0]
    chunks_per_tile = n_chunks // NW
    rows_per_tile = n // NS
    zrows = min(rows_per_tile, 128)

    mesh = plsc.VectorSubcoreMesh(core_axis_name="c", subcore_axis_name="s")

    @functools.partial(
        pl.kernel,
        out_type=jax.ShapeDtypeStruct((NC, n, d), jnp.float32),
        mesh=mesh,
        scratch_types=[
            pltpu.VMEM_SHARED((n, d), jnp.float32),
            pltpu.VMEM((zrows, d), jnp.float32),
            pltpu.VMEM((chunks_per_tile, CHUNK), jnp.int32),
            pltpu.VMEM((chunks_per_tile, CHUNK), jnp.int32),
            pltpu.VMEM((CHUNK, d), jnp.float32),
            pltpu.SemaphoreType.DMA,
        ],
    )
    def agg_kernel(x_hbm, src_hbm, dst_hbm, out_hbm, acc, zbuf, src_idx,
                   dst_idx, rows, gsem):
        cid = lax.axis_index("c")
        sid = lax.axis_index("s")
        wid = cid * NS + sid

        z16 = jnp.zeros((LANES,), jnp.float32)

        @pl.loop(0, zrows)
        def _(i):
            for j in range(d // LANES):
                zbuf[i, pl.ds(j * LANES, LANES)] = z16

        row0 = sid * rows_per_tile
        full, rem = divmod(rows_per_tile, zrows)
        for k in range(full):
            pltpu.sync_copy(zbuf, acc.at[pl.ds(row0 + k * zrows, zrows)])
        if rem:
            pltpu.sync_copy(zbuf.at[pl.ds(0, rem)],
                            acc.at[pl.ds(row0 + full * zrows, rem)])

        plsc.subcore_barrier()

        chunk0 = wid * chunks_per_tile
        pltpu.sync_copy(src_hbm.at[pl.ds(chunk0, chunks_per_tile)], src_idx)
        pltpu.sync_copy(dst_hbm.at[pl.ds(chunk0, chunks_per_tile)], dst_idx)

        @pl.loop(0, chunks_per_tile)
        def _(k):
            pltpu.async_copy(x_hbm.at[src_idx.at[k]], rows, gsem).wait()
            pltpu.sync_copy(rows, acc.at[dst_idx.at[k]], add=True)

        plsc.subcore_barrier()

        pltpu.sync_copy(acc.at[pl.ds(row0, rows_per_tile)],
                        out_hbm.at[cid, pl.ds(row0, rows_per_tile)])

    return agg_kernel(x, src2, dst2)


def _dense_block(p_ref, x_ref, wrel_ref, b_ref, wroot_ref, o_ref):
    agg = p_ref[0] + p_ref[1]
    rel = lax.dot_general(agg, wrel_ref[...], (((1,), (1,)), ((), ())),
                          preferred_element_type=jnp.float32)
    root = lax.dot_general(x_ref[...], wroot_ref[...], (((1,), (1,)), ((), ())),
                           preferred_element_type=jnp.float32)
    o_ref[...] = jnp.maximum(rel + b_ref[...] + root, 0.0)


@jax.jit
def _tc_dense(parts, x, wrel, brel, wroot):
    n, d = x.shape
    bn = 1000
    grid = n // bn
    return pl.pallas_call(
        _dense_block,
        grid=(grid,),
        in_specs=[
            pl.BlockSpec((NC, bn, d), lambda i: (0, i, 0)),
            pl.BlockSpec((bn, d), lambda i: (i, 0)),
            pl.BlockSpec((d, d), lambda i: (0, 0)),
            pl.BlockSpec((1, d), lambda i: (0, 0)),
            pl.BlockSpec((d, d), lambda i: (0, 0)),
        ],
        out_specs=pl.BlockSpec((bn, d), lambda i: (i, 0)),
        out_shape=jax.ShapeDtypeStruct((n, d), jnp.float32),
    )(parts, x, wrel, brel.reshape(1, d), wroot)


def kernel(x, edge_index, W1_rel, b1_rel, W1_root, W2_rel, b2_rel, W2_root):
    n, d = x.shape
    e = edge_index.shape[1]
    assert e % (NW * CHUNK) == 0 and n % NS == 0 and n % 1000 == 0

    src2 = edge_index[0].reshape(e // CHUNK, CHUNK)
    dst2 = edge_index[1].reshape(e // CHUNK, CHUNK)

    p1 = _sc_agg(x, src2, dst2, n_nodes=n)
    h = _tc_dense(p1, x, W1_rel, b1_rel, W1_root)
    p2 = _sc_agg(h, src2, dst2, n_nodes=n)
    return _tc_dense(p2, h, W2_rel, b2_rel, W2_root)


# speedup vs baseline: 2.8526x; 2.8526x over previous
import functools

import jax
import jax.numpy as jnp
from jax import lax
from jax.experimental import pallas as pl
from jax.experimental.pallas import tpu as pltpu
from jax.experimental.pallas import tpu_sc as plsc

NC = 2
NS = 16
LANES = 16
NW = NC * NS

CHUNK = 128


@functools.partial(jax.jit, static_argnames=("n_pad",))
def _sc_agg(x, src2, dst2, *, n_pad):
    _, d = x.shape
    n_chunks = src2.shape[0]
    chunks_per_tile = n_chunks // NW
    rows_per_tile = n_pad // NS

    mesh = plsc.VectorSubcoreMesh(core_axis_name="c", subcore_axis_name="s")

    @functools.partial(
        pl.kernel,
        out_type=jax.ShapeDtypeStruct((NC, n_pad, d), jnp.float32),
        mesh=mesh,
        scratch_types=[
            pltpu.VMEM_SHARED((n_pad, d), jnp.float32),
            pltpu.VMEM((chunks_per_tile, CHUNK), jnp.int32),
            pltpu.VMEM((chunks_per_tile, CHUNK), jnp.int32),
            pltpu.VMEM((CHUNK, d), jnp.float32),
            pltpu.SemaphoreType.DMA,
        ],
    )
    def agg_kernel(x_hbm, src_hbm, dst_hbm, out_hbm, acc, src_idx,
                   dst_idx, rows, gsem):
        cid = lax.axis_index("c")
        sid = lax.axis_index("s")
        wid = cid * NS + sid

        z16 = jnp.zeros((LANES,), jnp.float32)

        @pl.loop(0, CHUNK)
        def _(i):
            for j in range(d // LANES):
                rows[i, pl.ds(j * LANES, LANES)] = z16

        row0 = sid * rows_per_tile
        full, rem = divmod(rows_per_tile, CHUNK)
        for k in range(full):
            pltpu.sync_copy(rows, acc.at[pl.ds(row0 + k * CHUNK, CHUNK)])
        if rem:
            pltpu.sync_copy(rows.at[pl.ds(0, rem)],
                            acc.at[pl.ds(row0 + full * CHUNK, rem)])

        plsc.subcore_barrier()

        chunk0 = wid * chunks_per_tile
        pltpu.sync_copy(src_hbm.at[pl.ds(chunk0, chunks_per_tile)], src_idx)
        pltpu.sync_copy(dst_hbm.at[pl.ds(chunk0, chunks_per_tile)], dst_idx)

        @pl.loop(0, chunks_per_tile)
        def _(k):
            pltpu.async_copy(x_hbm.at[src_idx.at[k]], rows, gsem).wait()
            pltpu.sync_copy(rows, acc.at[dst_idx.at[k]], add=True)

        plsc.subcore_barrier()

        pltpu.sync_copy(acc.at[pl.ds(row0, rows_per_tile)],
                        out_hbm.at[cid, pl.ds(row0, rows_per_tile)])

    return agg_kernel(x, src2, dst2)


def _dense_block(p_ref, x_ref, wrel_ref, b_ref, wroot_ref, o_ref):
    agg = p_ref[0] + p_ref[1]
    rel = lax.dot_general(agg, wrel_ref[...], (((1,), (1,)), ((), ())),
                          preferred_element_type=jnp.float32)
    root = lax.dot_general(x_ref[...], wroot_ref[...], (((1,), (1,)), ((), ())),
                           preferred_element_type=jnp.float32)
    o_ref[...] = jnp.maximum(rel + b_ref[...] + root, 0.0)


@jax.jit
def _tc_dense(parts, x, wrel, brel, wroot):
    n, d = x.shape
    bn = 1000
    grid = n // bn
    return pl.pallas_call(
        _dense_block,
        grid=(grid,),
        in_specs=[
            pl.BlockSpec((NC, bn, d), lambda i: (0, i, 0)),
            pl.BlockSpec((bn, d), lambda i: (i, 0)),
            pl.BlockSpec((d, d), lambda i: (0, 0)),
            pl.BlockSpec((1, d), lambda i: (0, 0)),
            pl.BlockSpec((d, d), lambda i: (0, 0)),
        ],
        out_specs=pl.BlockSpec((bn, d), lambda i: (i, 0)),
        out_shape=jax.ShapeDtypeStruct((n, d), jnp.float32),
    )(parts, x, wrel, brel.reshape(1, d), wroot)


def kernel(x, edge_index, W1_rel, b1_rel, W1_root, W2_rel, b2_rel, W2_root):
    n, d = x.shape
    e = edge_index.shape[1]
    assert n % 1000 == 0 and d % LANES == 0

    n_pad = -(-n // (NS * 8)) * (NS * 8)
    e_pad = -(-e // (NW * CHUNK * 8)) * (NW * CHUNK * 8)
    pad = e_pad - e
    src = edge_index[0]
    dst = edge_index[1]
    if pad:
        src = jnp.concatenate([src, jnp.zeros((pad,), jnp.int32)])
        dst = jnp.concatenate([dst, jnp.full((pad,), n, jnp.int32)])
    src2 = src.reshape(e_pad // CHUNK, CHUNK)
    dst2 = dst.reshape(e_pad // CHUNK, CHUNK)

    p1 = _sc_agg(x, src2, dst2, n_pad=n_pad)
    h = _tc_dense(p1, x, W1_rel, b1_rel, W1_root)
    p2 = _sc_agg(h, src2, dst2, n_pad=n_pad)
    return _tc_dense(p2, h, W2_rel, b2_rel, W2_root)


# speedup vs baseline: 3.1101x; 1.0903x over previous
import functools

import jax
import jax.numpy as jnp
from jax import lax
from jax.experimental import pallas as pl
from jax.experimental.pallas import tpu as pltpu
from jax.experimental.pallas import tpu_sc as plsc

NC = 2
NS = 16
LANES = 16
NW = NC * NS

CHUNK = 128
SEG = 8


@functools.partial(jax.jit, static_argnames=("n_pad",))
def _sc_agg(x, src2, dst2, *, n_pad):
    _, d = x.shape
    n_chunks = src2.shape[0]
    chunks_per_tile = n_chunks // NW
    rows_per_tile = n_pad // NS
    n_seg = chunks_per_tile // SEG
    assert chunks_per_tile % SEG == 0 and n_seg % 2 == 0 and SEG % 2 == 0

    mesh = plsc.VectorSubcoreMesh(core_axis_name="c", subcore_axis_name="s")

    @functools.partial(
        pl.kernel,
        out_type=jax.ShapeDtypeStruct((NC, n_pad, d), jnp.float32),
        mesh=mesh,
        scratch_types=[
            pltpu.VMEM_SHARED((n_pad, d), jnp.float32),
            pltpu.VMEM((2, SEG, CHUNK), jnp.int32),
            pltpu.VMEM((2, SEG, CHUNK), jnp.int32),
            pltpu.VMEM((2, CHUNK, d), jnp.float32),
            pltpu.SemaphoreType.DMA,
            pltpu.SemaphoreType.DMA,
            pltpu.SemaphoreType.DMA,
            pltpu.SemaphoreType.DMA,
        ],
    )
    def agg_kernel(x_hbm, src_hbm, dst_hbm, out_hbm, acc, src_idx,
                   dst_idx, rows, gsem0, gsem1, isem0, isem1):
        cid = lax.axis_index("c")
        sid = lax.axis_index("s")
        wid = cid * NS + sid

        z16 = jnp.zeros((LANES,), jnp.float32)

        @pl.loop(0, CHUNK)
        def _(i):
            for j in range(d // LANES):
                rows[0, i, pl.ds(j * LANES, LANES)] = z16

        row0 = sid * rows_per_tile
        full, rem = divmod(rows_per_tile, CHUNK)
        for k in range(full):
            pltpu.sync_copy(rows.at[0], acc.at[pl.ds(row0 + k * CHUNK, CHUNK)])
        if rem:
            pltpu.sync_copy(rows.at[0, pl.ds(0, rem)],
                            acc.at[pl.ds(row0 + full * CHUNK, rem)])

        chunk0 = wid * chunks_per_tile
        gsems = (gsem0, gsem1)
        isems = (isem0, isem1)

        def stage_seg(s, p):
            off = chunk0 + s * SEG
            pltpu.async_copy(src_hbm.at[pl.ds(off, SEG)], src_idx.at[p],
                             isems[p])
            pltpu.async_copy(dst_hbm.at[pl.ds(off, SEG)], dst_idx.at[p],
                             isems[p])

        def wait_seg(s, p):
            off = chunk0 + s * SEG
            pltpu.make_async_copy(src_hbm.at[pl.ds(off, SEG)], src_idx.at[p],
                                  isems[p]).wait()
            pltpu.make_async_copy(dst_hbm.at[pl.ds(off, SEG)], dst_idx.at[p],
                                  isems[p]).wait()

        stage_seg(0, 0)
        wait_seg(0, 0)
        pltpu.async_copy(x_hbm.at[src_idx.at[0, 0]], rows.at[0], gsem0)
        stage_seg(1, 1)

        plsc.subcore_barrier()

        @pl.loop(0, n_seg, step=2)
        def _(s0):
            for sp in range(2):
                s = s0 + sp

                @pl.when(s > 0)
                def _():
                    wait_seg(s, sp)
                    pltpu.async_copy(x_hbm.at[src_idx.at[sp, 0]], rows.at[0],
                                     gsem0)

                @pl.loop(0, SEG, step=2)
                def _(c0):
                    for b in range(2):
                        c = c0 + b

                        @pl.when(c + 1 < SEG)
                        def _():
                            pltpu.async_copy(x_hbm.at[src_idx.at[sp, c + 1]],
                                             rows.at[1 - b], gsems[1 - b])

                        pltpu.make_async_copy(x_hbm.at[src_idx.at[sp, c]],
                                              rows.at[b], gsems[b]).wait()
                        pltpu.sync_copy(rows.at[b], acc.at[dst_idx.at[sp, c]],
                                        add=True)

                @pl.when(s + 2 < n_seg)
                def _():
                    stage_seg(s + 2, sp)

        plsc.subcore_barrier()

        pltpu.sync_copy(acc.at[pl.ds(row0, rows_per_tile)],
                        out_hbm.at[cid, pl.ds(row0, rows_per_tile)])

    return agg_kernel(x, src2, dst2)


def _dense_block(p_ref, x_ref, wrel_ref, b_ref, wroot_ref, o_ref):
    agg = p_ref[0] + p_ref[1]
    rel = lax.dot_general(agg, wrel_ref[...], (((1,), (1,)), ((), ())),
                          preferred_element_type=jnp.float32)
    root = lax.dot_general(x_ref[...], wroot_ref[...], (((1,), (1,)), ((), ())),
                           preferred_element_type=jnp.float32)
    o_ref[...] = jnp.maximum(rel + b_ref[...] + root, 0.0)


@jax.jit
def _tc_dense(parts, x, wrel, brel, wroot):
    n, d = x.shape
    bn = 1000
    grid = n // bn
    return pl.pallas_call(
        _dense_block,
        grid=(grid,),
        in_specs=[
            pl.BlockSpec((NC, bn, d), lambda i: (0, i, 0)),
            pl.BlockSpec((bn, d), lambda i: (i, 0)),
            pl.BlockSpec((d, d), lambda i: (0, 0)),
            pl.BlockSpec((1, d), lambda i: (0, 0)),
            pl.BlockSpec((d, d), lambda i: (0, 0)),
        ],
        out_specs=pl.BlockSpec((bn, d), lambda i: (i, 0)),
        out_shape=jax.ShapeDtypeStruct((n, d), jnp.float32),
    )(parts, x, wrel, brel.reshape(1, d), wroot)


def kernel(x, edge_index, W1_rel, b1_rel, W1_root, W2_rel, b2_rel, W2_root):
    n, d = x.shape
    e = edge_index.shape[1]
    assert n % 1000 == 0 and d % LANES == 0

    n_pad = -(-n // (NS * 8)) * (NS * 8)
    e_pad = -(-e // (NW * CHUNK * 8)) * (NW * CHUNK * 8)
    pad = e_pad - e
    src = edge_index[0]
    dst = edge_index[1]
    if pad:
        src = jnp.concatenate([src, jnp.zeros((pad,), jnp.int32)])
        dst = jnp.concatenate([dst, jnp.full((pad,), n, jnp.int32)])
    src2 = src.reshape(e_pad // CHUNK, CHUNK)
    dst2 = dst.reshape(e_pad // CHUNK, CHUNK)

    p1 = _sc_agg(x, src2, dst2, n_pad=n_pad)
    h = _tc_dense(p1, x, W1_rel, b1_rel, W1_root)
    p2 = _sc_agg(h, src2, dst2, n_pad=n_pad)
    return _tc_dense(p2, h, W2_rel, b2_rel, W2_root)


# speedup vs baseline: 12.0228x; 3.8657x over previous
import functools

import jax
import jax.numpy as jnp
from jax import lax
from jax.experimental import pallas as pl
from jax.experimental.pallas import tpu as pltpu
from jax.experimental.pallas import tpu_sc as plsc

NC = 2
NS = 16
LANES = 16
NW = NC * NS

CHUNK = 128
SEG = 8


@functools.partial(jax.jit, static_argnames=("n_pad",))
def _sc_agg(x, src2, dst2, *, n_pad):
    _, d = x.shape
    n_chunks = src2.shape[0]
    chunks_per_tile = n_chunks // NW
    rows_per_tile = n_pad // NS
    n_seg = chunks_per_tile // SEG
    assert chunks_per_tile % SEG == 0 and n_seg % 2 == 0 and SEG % 2 == 0

    mesh = plsc.VectorSubcoreMesh(core_axis_name="c", subcore_axis_name="s")

    @functools.partial(
        pl.kernel,
        out_type=jax.ShapeDtypeStruct((NC, n_pad, d), jnp.float32),
        mesh=mesh,
        scratch_types=[
            pltpu.VMEM_SHARED((n_pad, d), jnp.float32),
            pltpu.VMEM((2, SEG, CHUNK), jnp.int32),
            pltpu.VMEM((2, SEG, CHUNK), jnp.int32),
            pltpu.VMEM((2, CHUNK, d), jnp.float32),
            pltpu.SemaphoreType.DMA,
            pltpu.SemaphoreType.DMA,
            pltpu.SemaphoreType.DMA,
            pltpu.SemaphoreType.DMA,
        ],
    )
    def agg_kernel(x_hbm, src_hbm, dst_hbm, out_hbm, acc, src_idx,
                   dst_idx, rows, gsem0, gsem1, isem0, isem1):
        cid = lax.axis_index("c")
        sid = lax.axis_index("s")
        wid = cid * NS + sid

        z16 = jnp.zeros((LANES,), jnp.float32)

        @pl.loop(0, CHUNK)
        def _(i):
            for j in range(d // LANES):
                rows[0, i, pl.ds(j * LANES, LANES)] = z16

        row0 = sid * rows_per_tile
        full, rem = divmod(rows_per_tile, CHUNK)
        for k in range(full):
            pltpu.sync_copy(rows.at[0], acc.at[pl.ds(row0 + k * CHUNK, CHUNK)])
        if rem:
            pltpu.sync_copy(rows.at[0, pl.ds(0, rem)],
                            acc.at[pl.ds(row0 + full * CHUNK, rem)])

        chunk0 = wid * chunks_per_tile
        gsems = (gsem0, gsem1)
        isems = (isem0, isem1)

        def stage_seg(s, p):
            off = chunk0 + s * SEG
            pltpu.async_copy(src_hbm.at[pl.ds(off, SEG)], src_idx.at[p],
                             isems[p])
            pltpu.async_copy(dst_hbm.at[pl.ds(off, SEG)], dst_idx.at[p],
                             isems[p])

        def wait_seg(s, p):
            off = chunk0 + s * SEG
            pltpu.make_async_copy(src_hbm.at[pl.ds(off, SEG)], src_idx.at[p],
                                  isems[p]).wait()
            pltpu.make_async_copy(dst_hbm.at[pl.ds(off, SEG)], dst_idx.at[p],
                                  isems[p]).wait()

        stage_seg(0, 0)
        wait_seg(0, 0)
        pltpu.async_copy(x_hbm.at[src_idx.at[0, 0]], rows.at[0], gsem0)
        stage_seg(1, 1)

        plsc.subcore_barrier()

        @pl.loop(0, n_seg, step=2)
        def _(s0):
            for sp in range(2):
                s = s0 + sp

                @pl.when(s > 0)
                def _():
                    wait_seg(s, sp)
                    pltpu.async_copy(x_hbm.at[src_idx.at[sp, 0]], rows.at[0],
                                     gsem0)

                @pl.loop(0, SEG, step=2)
                def _(c0):
                    for b in range(2):
                        c = c0 + b

                        @pl.when(c + 1 < SEG)
                        def _():
                            pltpu.async_copy(x_hbm.at[src_idx.at[sp, c + 1]],
                                             rows.at[1 - b], gsems[1 - b])

                        pltpu.make_async_copy(x_hbm.at[src_idx.at[sp, c]],
                                              rows.at[b], gsems[b]).wait()
                        pltpu.sync_copy(rows.at[b], acc.at[dst_idx.at[sp, c]],
                                        add=True)

                @pl.when(s + 2 < n_seg)
                def _():
                    stage_seg(s + 2, sp)

        plsc.subcore_barrier()

        pltpu.sync_copy(acc.at[pl.ds(row0, rows_per_tile)],
                        out_hbm.at[cid, pl.ds(row0, rows_per_tile)])

    return agg_kernel(x, src2, dst2)


def _dense_block(p_ref, x_ref, wrel_ref, b_ref, wroot_ref, o_ref):
    agg = p_ref[0] + p_ref[1]
    rel = lax.dot_general(agg, wrel_ref[...], (((1,), (1,)), ((), ())),
                          preferred_element_type=jnp.float32)
    root = lax.dot_general(x_ref[...], wroot_ref[...], (((1,), (1,)), ((), ())),
                           preferred_element_type=jnp.float32)
    o_ref[...] = jnp.maximum(rel + b_ref[...] + root, 0.0)


@jax.jit
def _tc_dense(parts, x, wrel, brel, wroot):
    n, d = x.shape
    bn = 1000
    grid = n // bn
    return pl.pallas_call(
        _dense_block,
        grid=(grid,),
        in_specs=[
            pl.BlockSpec((NC, bn, d), lambda i: (0, i, 0)),
            pl.BlockSpec((bn, d), lambda i: (i, 0)),
            pl.BlockSpec((d, d), lambda i: (0, 0)),
            pl.BlockSpec((1, d), lambda i: (0, 0)),
            pl.BlockSpec((d, d), lambda i: (0, 0)),
        ],
        out_specs=pl.BlockSpec((bn, d), lambda i: (i, 0)),
        out_shape=jax.ShapeDtypeStruct((n, d), jnp.float32),
    )(parts, x, wrel, brel.reshape(1, d), wroot)


def kernel(x, edge_index, W1_rel, b1_rel, W1_root, W2_rel, b2_rel, W2_root):
    n, d = x.shape
    e = edge_index.shape[1]
    assert n % 1000 == 0 and d % LANES == 0

    n_pad = -(-n // (NS * 8)) * (NS * 8)
    e_pad = -(-e // (NW * CHUNK * 8)) * (NW * CHUNK * 8)
    pad = e_pad - e
    src = edge_index[0]
    dst = edge_index[1]
    if pad:
        src = jnp.concatenate([src, jnp.arange(pad, dtype=jnp.int32) % n])
        dst = jnp.concatenate(
            [dst, n + jnp.arange(pad, dtype=jnp.int32) % (n_pad - n)])
    src2 = src.reshape(e_pad // CHUNK, CHUNK)
    dst2 = dst.reshape(e_pad // CHUNK, CHUNK)

    p1 = _sc_agg(x, src2, dst2, n_pad=n_pad)
    h = _tc_dense(p1, x, W1_rel, b1_rel, W1_root)
    p2 = _sc_agg(h, src2, dst2, n_pad=n_pad)
    return _tc_dense(p2, h, W2_rel, b2_rel, W2_root)
